# baseline (speedup 1.0000x reference)
import jax
import jax.numpy as jnp
from jax import lax
from jax.experimental import pallas as pl
from jax.experimental.pallas import tpu as pltpu
from jax.experimental.pallas import tpu_sc as plsc

_J = 17
_L = 16
_NG = 2


def _sc_body(x_hbm, out_hbm, x_v, o_v):
    wid = lax.axis_index("s") * 2 + lax.axis_index("c")

    @pl.when(wid < _NG)
    def _():
        pltpu.sync_copy(x_hbm.at[wid], x_v)
        best = x_v[0, :]
        bidx = jnp.zeros((_L,), jnp.int32)
        for j in range(1, _J):
            v = x_v[j, :]
            gt = v > best
            best = jnp.where(gt, v, best)
            bidx = jnp.where(gt, j, bidx)
        for j in range(_J):
            v = x_v[j, :]
            y = jnp.where(bidx == j, jnp.float32(1.0), jnp.float32(0.0))
            o_v[j, :] = (y - v) + v
        pltpu.sync_copy(o_v, out_hbm.at[wid])


def kernel(sgt_trans_mat, use_gumbel_noise, gumbel_temp):
    x = sgt_trans_mat
    hw = x.shape[1]
    xt = jnp.swapaxes(x[0], 0, 1)
    xt = jnp.pad(xt, ((0, 0), (0, _NG * _L - hw)))
    xg = jnp.swapaxes(xt.reshape(_J, _NG, _L), 0, 1)

    mesh = plsc.VectorSubcoreMesh(core_axis_name="c", subcore_axis_name="s")
    out = pl.kernel(
        _sc_body,
        out_type=jax.ShapeDtypeStruct((_NG, _J, _L), jnp.float32),
        mesh=mesh,
        scratch_types=[
            pltpu.VMEM((_J, _L), jnp.float32),
            pltpu.VMEM((_J, _L), jnp.float32),
        ],
    )(xg)

    og = jnp.swapaxes(out, 0, 1).reshape(_J, _NG * _L)[:, :hw]
    return jnp.swapaxes(og, 0, 1)[None]


# speedup vs baseline: 1.0762x; 1.0762x over previous
import jax
import jax.numpy as jnp
from jax import lax
from jax.experimental import pallas as pl
from jax.experimental.pallas import tpu as pltpu
from jax.experimental.pallas import tpu_sc as plsc

_J = 17
_L = 16
_NG = 2


def _sc_body(x_hbm, out_hbm, x_v, o_v):
    pltpu.sync_copy(x_hbm, x_v)
    for g in range(_NG):
        best = x_v[g, 0, :]
        bidx = jnp.zeros((_L,), jnp.int32)
        for j in range(1, _J):
            v = x_v[g, j, :]
            gt = v > best
            best = jnp.where(gt, v, best)
            bidx = jnp.where(gt, j, bidx)
        for j in range(_J):
            v = x_v[g, j, :]
            y = jnp.where(bidx == j, jnp.float32(1.0), jnp.float32(0.0))
            o_v[g, j, :] = (y - v) + v
    pltpu.sync_copy(o_v, out_hbm)


def kernel(sgt_trans_mat, use_gumbel_noise, gumbel_temp):
    x = sgt_trans_mat
    hw = x.shape[1]
    xt = jnp.swapaxes(x[0], 0, 1)
    xt = jnp.pad(xt, ((0, 0), (0, _NG * _L - hw)))
    xg = jnp.swapaxes(xt.reshape(_J, _NG, _L), 0, 1)

    mesh = plsc.VectorSubcoreMesh(core_axis_name="c", subcore_axis_name="s",
                                  num_cores=1, num_subcores=1)
    out = pl.kernel(
        _sc_body,
        out_type=jax.ShapeDtypeStruct((_NG, _J, _L), jnp.float32),
        mesh=mesh,
        scratch_types=[
            pltpu.VMEM((_NG, _J, _L), jnp.float32),
            pltpu.VMEM((_NG, _J, _L), jnp.float32),
        ],
    )(xg)

    og = jnp.swapaxes(out, 0, 1).reshape(_J, _NG * _L)[:, :hw]
    return jnp.swapaxes(og, 0, 1)[None]


# speedup vs baseline: 1.0888x; 1.0117x over previous
import jax
import jax.numpy as jnp
from jax.experimental import pallas as pl
from jax.experimental.pallas import tpu as pltpu
from jax.experimental.pallas import tpu_sc as plsc

_J = 17
_L = 16
_NG = 2
_HW = 25


def _sc_body(x_hbm, out_hbm, x_v, o_v):
    pltpu.sync_copy(x_hbm, x_v)
    for g in range(_NG):
        sl = pl.ds(g * _L, _L)
        best = x_v[0, sl]
        bidx = jnp.zeros((_L,), jnp.int32)
        for j in range(1, _J):
            v = x_v[j, sl]
            gt = v > best
            best = jnp.where(gt, v, best)
            bidx = jnp.where(gt, j, bidx)
        for j in range(_J):
            v = x_v[j, sl]
            y = jnp.where(bidx == j, jnp.float32(1.0), jnp.float32(0.0))
            o_v[j, sl] = (y - v) + v
    pltpu.sync_copy(o_v, out_hbm)


def kernel(sgt_trans_mat, use_gumbel_noise, gumbel_temp):
    x = sgt_trans_mat
    xp = jnp.pad(x[0], ((0, _NG * _L - _HW), (0, 0)))
    xt = jnp.swapaxes(xp, 0, 1)

    mesh = plsc.VectorSubcoreMesh(core_axis_name="c", subcore_axis_name="s",
                                  num_cores=1, num_subcores=1)
    out = pl.kernel(
        _sc_body,
        out_type=jax.ShapeDtypeStruct((_J, _NG * _L), jnp.float32),
        mesh=mesh,
        scratch_types=[
            pltpu.VMEM((_J, _NG * _L), jnp.float32),
            pltpu.VMEM((_J, _NG * _L), jnp.float32),
        ],
    )(xt)

    return jnp.swapaxes(out, 0, 1)[None, :_HW, :]
